# Initial kernel scaffold; baseline (speedup 1.0000x reference)
import jax
import jax.numpy as jnp
from jax.experimental import pallas as pl


def kernel(metadata, deltas, proposals, scores, masks):
    raise NotImplementedError("write your pallas kernel here")


# speedup vs baseline: 76.0750x; 76.0750x over previous
import functools

import jax
import jax.numpy as jnp
from jax.experimental import pallas as pl
from jax.experimental.pallas import tpu as pltpu

_N = 5000
_C = 21
_MH = 14
_MW = 14
_PAD = 300
_ROWS = 40
_LANES = 128
_NP = _ROWS * _LANES
_GPAD = 320
_D = _MH * _MW * _C


def _nms_body(meta_ref, x1_ref, y1_ref, x2_ref, y2_ref, sc_ref, dl_ref,
              boxes_ref, idx_ref, cnt_ref):
    scale = meta_ref[0, 2]
    h_img = meta_ref[0, 0]
    w_img = meta_ref[0, 1]
    x1 = x1_ref[...] / scale
    y1 = y1_ref[...] / scale
    x2 = x2_ref[...] / scale
    y2 = y2_ref[...] / scale
    wa = x2 - x1 + 1.0
    ha = y2 - y1 + 1.0
    cxa = x1 + 0.5 * wa
    cya = y1 + 0.5 * ha

    best = sc_ref[0]
    top = jnp.zeros((_ROWS, _LANES), jnp.int32)
    maxsc = sc_ref[1]
    for c in range(1, _C):
        plane = sc_ref[c]
        top = jnp.where(plane > best, c, top)
        best = jnp.maximum(best, plane)
        if c > 1:
            maxsc = jnp.maximum(maxsc, plane)

    dx = dl_ref[0]
    dy = dl_ref[1]
    dw = dl_ref[2]
    dh = dl_ref[3]
    for c in range(1, _C):
        sel = top == c
        dx = jnp.where(sel, dl_ref[4 * c + 0], dx)
        dy = jnp.where(sel, dl_ref[4 * c + 1], dy)
        dw = jnp.where(sel, dl_ref[4 * c + 2], dw)
        dh = jnp.where(sel, dl_ref[4 * c + 3], dh)

    cx = dx * wa + cxa
    cy = dy * ha + cya
    w = jnp.exp(dw) * wa
    h = jnp.exp(dh) * ha
    px1 = jnp.clip(cx - 0.5 * w, 0.0, w_img - 1.0)
    py1 = jnp.clip(cy - 0.5 * h, 0.0, h_img - 1.0)
    px2 = jnp.clip(cx + 0.5 * w, 0.0, w_img - 1.0)
    py2 = jnp.clip(cy + 0.5 * h, 0.0, h_img - 1.0)
    areas = (px2 - px1 + 1.0) * (py2 - py1 + 1.0)

    row = jax.lax.broadcasted_iota(jnp.int32, (_ROWS, _LANES), 0)
    lane = jax.lax.broadcasted_iota(jnp.int32, (_ROWS, _LANES), 1)
    fidx = row * _LANES + lane
    cand0 = jnp.where(fidx < _N, 1.0, 0.0).astype(jnp.float32)
    neg_inf = jnp.float32(-jnp.inf)

    def cond(state):
        k, go, _ = state
        return (go > 0) & (k < _PAD)

    def body(state):
        k, _, candf = state
        cand = candf > 0.0
        masked = jnp.where(cand, maxsc, neg_inf)
        m_val = jnp.max(masked)
        is_m = (masked == m_val) & cand
        m = jnp.min(jnp.where(is_m, fidx, jnp.int32(2**30)))
        sel = fidx == m
        bx1 = jnp.sum(jnp.where(sel, px1, 0.0))
        by1 = jnp.sum(jnp.where(sel, py1, 0.0))
        bx2 = jnp.sum(jnp.where(sel, px2, 0.0))
        by2 = jnp.sum(jnp.where(sel, py2, 0.0))
        ba = jnp.sum(jnp.where(sel, areas, 0.0))
        iw = jnp.maximum(jnp.minimum(bx2, px2) - jnp.maximum(bx1, px1) + 1.0, 0.0)
        ih = jnp.maximum(jnp.minimum(by2, py2) - jnp.maximum(by1, py1) + 1.0, 0.0)
        inter = iw * ih
        iou = inter / (ba + areas - inter)
        newcandf = jnp.where(cand & ~((iou > 0.5) | sel), 1.0, 0.0).astype(
            jnp.float32)
        idx_ref[k] = m
        boxes_ref[k, 0] = bx1
        boxes_ref[k, 1] = by1
        boxes_ref[k, 2] = bx2
        boxes_ref[k, 3] = by2
        go = jnp.where(jnp.max(newcandf) > 0.0, 1, 0).astype(jnp.int32)
        return k + 1, go, newcandf

    kfin, _, _ = jax.lax.while_loop(
        cond, body, (jnp.int32(0), jnp.int32(1), cand0))
    cnt_ref[0] = kfin

    def ztail(i, carry):
        @pl.when(i >= kfin)
        def _():
            idx_ref[i] = 0
            boxes_ref[i, 0] = 0.0
            boxes_ref[i, 1] = 0.0
            boxes_ref[i, 2] = 0.0
            boxes_ref[i, 3] = 0.0
        return carry

    jax.lax.fori_loop(0, _PAD, ztail, 0)
    for i in range(_PAD, _GPAD):
        idx_ref[i] = 0


def _gather_body(idx_ref, cnt_ref, masks_ref, sc_ref, masks_out_ref, sc_out_ref):
    i = pl.program_id(0)
    v = jnp.where(i < cnt_ref[0], jnp.float32(1.0), jnp.float32(0.0))
    masks_out_ref[...] = masks_ref[...] * v
    sc_out_ref[...] = sc_ref[...] * v


@jax.jit
def kernel(metadata, deltas, proposals, scores, masks):
    p = proposals.reshape(_N, 4)
    pad = _NP - _N
    planes = [jnp.pad(p[:, k], (0, pad)).reshape(_ROWS, _LANES) for k in range(4)]
    sc = scores.reshape(_N, _C)
    sct = jnp.pad(sc.T, ((0, 0), (0, pad))).reshape(_C, _ROWS, _LANES)
    dlt = jnp.pad(deltas.reshape(_N, 4 * _C).T, ((0, 0), (0, pad))).reshape(
        4 * _C, _ROWS, _LANES)

    boxes, idx, cnt = pl.pallas_call(
        _nms_body,
        out_shape=[
            jax.ShapeDtypeStruct((_PAD, 4), jnp.float32),
            jax.ShapeDtypeStruct((_GPAD,), jnp.int32),
            jax.ShapeDtypeStruct((1,), jnp.int32),
        ],
        in_specs=[
            pl.BlockSpec(memory_space=pltpu.SMEM),
            pl.BlockSpec(memory_space=pltpu.VMEM),
            pl.BlockSpec(memory_space=pltpu.VMEM),
            pl.BlockSpec(memory_space=pltpu.VMEM),
            pl.BlockSpec(memory_space=pltpu.VMEM),
            pl.BlockSpec(memory_space=pltpu.VMEM),
            pl.BlockSpec(memory_space=pltpu.VMEM),
        ],
        out_specs=[
            pl.BlockSpec(memory_space=pltpu.SMEM),
            pl.BlockSpec(memory_space=pltpu.SMEM),
            pl.BlockSpec(memory_space=pltpu.SMEM),
        ],
    )(metadata, *planes, sct, dlt)

    masks_tab = masks.reshape(_N, 1, _D)
    sc_tab = jnp.pad(sc, ((0, 0), (0, 32 - _C))).reshape(_N, 1, 32)
    grid_spec = pltpu.PrefetchScalarGridSpec(
        num_scalar_prefetch=2,
        grid=(_PAD,),
        in_specs=[
            pl.BlockSpec((1, 1, _D), lambda i, idx, cnt: (idx[i], 0, 0)),
            pl.BlockSpec((1, 1, 32), lambda i, idx, cnt: (idx[i], 0, 0)),
        ],
        out_specs=[
            pl.BlockSpec((1, 1, _D), lambda i, idx, cnt: (i, 0, 0)),
            pl.BlockSpec((1, 1, 32), lambda i, idx, cnt: (i, 0, 0)),
        ],
    )
    masks_out, sc_out = pl.pallas_call(
        _gather_body,
        grid_spec=grid_spec,
        out_shape=[
            jax.ShapeDtypeStruct((_PAD, 1, _D), jnp.float32),
            jax.ShapeDtypeStruct((_PAD, 1, 32), jnp.float32),
        ],
    )(idx[:_PAD], cnt, masks_tab, sc_tab)

    out_boxes = boxes[None]
    out_scores = sc_out.reshape(_PAD, 32)[:, :_C][None]
    out_masks = masks_out.reshape(_PAD, _MH, _MW, _C)[None]
    return out_boxes, out_scores, out_masks


# speedup vs baseline: 85.1761x; 1.1196x over previous
import functools

import jax
import jax.numpy as jnp
from jax.experimental import pallas as pl
from jax.experimental.pallas import tpu as pltpu

_N = 5000
_C = 21
_MH = 14
_MW = 14
_PAD = 300
_ROWS = 40
_LANES = 128
_NP = _ROWS * _LANES
_GPAD = 320
_D = _MH * _MW * _C
_BIG = 2**30


def _nms_body(meta_ref, x1_ref, y1_ref, x2_ref, y2_ref, sc_ref, dl_ref,
              pb_ref, idx_ref, cnt_ref):
    scale = meta_ref[0, 2]
    h_img = meta_ref[0, 0]
    w_img = meta_ref[0, 1]
    x1 = x1_ref[...] / scale
    y1 = y1_ref[...] / scale
    x2 = x2_ref[...] / scale
    y2 = y2_ref[...] / scale
    wa = x2 - x1 + 1.0
    ha = y2 - y1 + 1.0
    cxa = x1 + 0.5 * wa
    cya = y1 + 0.5 * ha

    best = sc_ref[0]
    top = jnp.zeros((_ROWS, _LANES), jnp.int32)
    maxsc = sc_ref[1]
    for c in range(1, _C):
        plane = sc_ref[c]
        top = jnp.where(plane > best, c, top)
        best = jnp.maximum(best, plane)
        if c > 1:
            maxsc = jnp.maximum(maxsc, plane)

    dx = dl_ref[0]
    dy = dl_ref[1]
    dw = dl_ref[2]
    dh = dl_ref[3]
    for c in range(1, _C):
        sel = top == c
        dx = jnp.where(sel, dl_ref[4 * c + 0], dx)
        dy = jnp.where(sel, dl_ref[4 * c + 1], dy)
        dw = jnp.where(sel, dl_ref[4 * c + 2], dw)
        dh = jnp.where(sel, dl_ref[4 * c + 3], dh)

    cx = dx * wa + cxa
    cy = dy * ha + cya
    w = jnp.exp(dw) * wa
    h = jnp.exp(dh) * ha
    px1 = jnp.clip(cx - 0.5 * w, 0.0, w_img - 1.0)
    py1 = jnp.clip(cy - 0.5 * h, 0.0, h_img - 1.0)
    px2 = jnp.clip(cx + 0.5 * w, 0.0, w_img - 1.0)
    py2 = jnp.clip(cy + 0.5 * h, 0.0, h_img - 1.0)
    areas = (px2 - px1 + 1.0) * (py2 - py1 + 1.0)
    pb_ref[0] = px1
    pb_ref[1] = py1
    pb_ref[2] = px2
    pb_ref[3] = py2

    row = jax.lax.broadcasted_iota(jnp.int32, (_ROWS, _LANES), 0)
    lane = jax.lax.broadcasted_iota(jnp.int32, (_ROWS, _LANES), 1)
    fidx = row * _LANES + lane
    cand0 = jnp.where(fidx < _N, 1.0, 0.0).astype(jnp.float32)
    neg_inf = jnp.float32(-jnp.inf)

    def _red(op, x):
        return op(op(x, axis=1, keepdims=True), axis=0, keepdims=True)

    def body(k, state):
        candf, cnt = state
        cand = candf > 0.0
        masked = jnp.where(cand, maxsc, neg_inf)
        m_val = _red(jnp.max, masked)
        is_m = (masked == m_val) & cand
        m_idx = _red(jnp.min, jnp.where(is_m, fidx, _BIG))
        sel = fidx == m_idx
        bx1 = _red(jnp.sum, jnp.where(sel, px1, 0.0))
        by1 = _red(jnp.sum, jnp.where(sel, py1, 0.0))
        bx2 = _red(jnp.sum, jnp.where(sel, px2, 0.0))
        by2 = _red(jnp.sum, jnp.where(sel, py2, 0.0))
        ba = _red(jnp.sum, jnp.where(sel, areas, 0.0))
        iw = jnp.maximum(jnp.minimum(bx2, px2) - jnp.maximum(bx1, px1) + 1.0,
                         0.0)
        ih = jnp.maximum(jnp.minimum(by2, py2) - jnp.maximum(by1, py1) + 1.0,
                         0.0)
        inter = iw * ih
        iou = inter / (ba + areas - inter)
        newcandf = jnp.where(cand & ~((iou > 0.5) | sel), 1.0, 0.0).astype(
            jnp.float32)
        m_scalar = m_idx[0, 0]
        alive = m_scalar < _BIG
        idx_ref[k] = jnp.where(alive, m_scalar, 0)
        return newcandf, cnt + jnp.where(alive, 1, 0).astype(jnp.int32)

    _, kfin = jax.lax.fori_loop(0, _PAD, body, (cand0, jnp.int32(0)))
    cnt_ref[0] = kfin
    for i in range(_PAD, _GPAD):
        idx_ref[i] = 0


def _gather_body(idx_ref, cnt_ref, pb_ref, masks_ref, sc_ref,
                 pb_out_ref, masks_out_ref, sc_out_ref):
    i = pl.program_id(0)
    v = jnp.where(i < cnt_ref[0], jnp.float32(1.0), jnp.float32(0.0))
    pb_out_ref[...] = pb_ref[...] * v
    masks_out_ref[...] = masks_ref[...] * v
    sc_out_ref[...] = sc_ref[...] * v


@jax.jit
def kernel(metadata, deltas, proposals, scores, masks):
    p = proposals.reshape(_N, 4)
    pad = _NP - _N
    planes = [jnp.pad(p[:, k], (0, pad)).reshape(_ROWS, _LANES) for k in range(4)]
    sc = scores.reshape(_N, _C)
    sct = jnp.pad(sc.T, ((0, 0), (0, pad))).reshape(_C, _ROWS, _LANES)
    dlt = jnp.pad(deltas.reshape(_N, 4 * _C).T, ((0, 0), (0, pad))).reshape(
        4 * _C, _ROWS, _LANES)

    pb, idx, cnt = pl.pallas_call(
        _nms_body,
        out_shape=[
            jax.ShapeDtypeStruct((4, _ROWS, _LANES), jnp.float32),
            jax.ShapeDtypeStruct((_GPAD,), jnp.int32),
            jax.ShapeDtypeStruct((1,), jnp.int32),
        ],
        in_specs=[
            pl.BlockSpec(memory_space=pltpu.SMEM),
            pl.BlockSpec(memory_space=pltpu.VMEM),
            pl.BlockSpec(memory_space=pltpu.VMEM),
            pl.BlockSpec(memory_space=pltpu.VMEM),
            pl.BlockSpec(memory_space=pltpu.VMEM),
            pl.BlockSpec(memory_space=pltpu.VMEM),
            pl.BlockSpec(memory_space=pltpu.VMEM),
        ],
        out_specs=[
            pl.BlockSpec(memory_space=pltpu.VMEM),
            pl.BlockSpec(memory_space=pltpu.SMEM),
            pl.BlockSpec(memory_space=pltpu.SMEM),
        ],
    )(metadata, *planes, sct, dlt)

    pb_tab = pb.reshape(4, _NP).T.reshape(_NP, 1, 4)
    masks_tab = masks.reshape(_N, 1, _D)
    sc_tab = jnp.pad(sc, ((0, 0), (0, 32 - _C))).reshape(_N, 1, 32)
    grid_spec = pltpu.PrefetchScalarGridSpec(
        num_scalar_prefetch=2,
        grid=(_PAD,),
        in_specs=[
            pl.BlockSpec((1, 1, 4), lambda i, idx, cnt: (idx[i], 0, 0)),
            pl.BlockSpec((1, 1, _D), lambda i, idx, cnt: (idx[i], 0, 0)),
            pl.BlockSpec((1, 1, 32), lambda i, idx, cnt: (idx[i], 0, 0)),
        ],
        out_specs=[
            pl.BlockSpec((1, 1, 4), lambda i, idx, cnt: (i, 0, 0)),
            pl.BlockSpec((1, 1, _D), lambda i, idx, cnt: (i, 0, 0)),
            pl.BlockSpec((1, 1, 32), lambda i, idx, cnt: (i, 0, 0)),
        ],
    )
    pb_out, masks_out, sc_out = pl.pallas_call(
        _gather_body,
        grid_spec=grid_spec,
        out_shape=[
            jax.ShapeDtypeStruct((_PAD, 1, 4), jnp.float32),
            jax.ShapeDtypeStruct((_PAD, 1, _D), jnp.float32),
            jax.ShapeDtypeStruct((_PAD, 1, 32), jnp.float32),
        ],
    )(idx[:_PAD], cnt, pb_tab, masks_tab, sc_tab)

    out_boxes = pb_out.reshape(_PAD, 4)[None]
    out_scores = sc_out.reshape(_PAD, 32)[:, :_C][None]
    out_masks = masks_out.reshape(_PAD, _MH, _MW, _C)[None]
    return out_boxes, out_scores, out_masks
